# Initial kernel scaffold; baseline (speedup 1.0000x reference)
import jax
import jax.numpy as jnp
from jax.experimental import pallas as pl


def kernel(tokens, table):
    raise NotImplementedError("write your pallas kernel here")


# speedup vs baseline: 3.4650x; 3.4650x over previous
import functools
import math

import jax
import jax.numpy as jnp
from jax import lax
from jax.experimental import pallas as pl
from jax.experimental.pallas import tpu as pltpu
from jax.experimental.pallas import tpu_sc as plsc

EMB = 64
SCALE = math.sqrt(EMB)

NC = 2
NS = 16
NW = NC * NS

CHUNK = 128


def _scale_body(x_ref, o_ref):
    o_ref[...] = x_ref[...] * SCALE


def _scale_table(table):
    v, d = table.shape
    n = v * d
    cols = 128
    rows = n // cols
    block_rows = rows // 10
    t2 = table.reshape(rows, cols)
    scaled = pl.pallas_call(
        _scale_body,
        out_shape=jax.ShapeDtypeStruct((rows, cols), jnp.float32),
        grid=(10,),
        in_specs=[pl.BlockSpec((block_rows, cols), lambda i: (i, 0))],
        out_specs=pl.BlockSpec((block_rows, cols), lambda i: (i, 0)),
    )(t2)
    return scaled.reshape(v, d)


@functools.partial(jax.jit, static_argnames=("n_chunks",))
def _sc_gather(idx3, table_scaled, *, n_chunks):
    b_total = NW * n_chunks * CHUNK
    b_per_w = n_chunks * CHUNK
    mesh = plsc.VectorSubcoreMesh(core_axis_name="c", subcore_axis_name="s")

    @functools.partial(
        pl.kernel,
        out_type=jax.ShapeDtypeStruct((b_total, EMB), jnp.float32),
        mesh=mesh,
        scratch_types=[
            pltpu.VMEM((n_chunks, CHUNK), jnp.int32),
            pltpu.VMEM((CHUNK, EMB), jnp.float32),
            pltpu.SemaphoreType.DMA,
        ],
        compiler_params=pltpu.CompilerParams(use_tc_tiling_on_sc=False),
    )
    def k(idx_hbm, tab_hbm, out_hbm, idx_v, rows_v, gsem):
        wid = lax.axis_index("s") * NC + lax.axis_index("c")
        base = wid * b_per_w
        pltpu.sync_copy(idx_hbm.at[wid], idx_v)

        @pl.loop(0, n_chunks)
        def _chunk(j):
            pltpu.async_copy(tab_hbm.at[idx_v.at[j]], rows_v, gsem).wait()
            pltpu.sync_copy(rows_v, out_hbm.at[pl.ds(base + j * CHUNK, CHUNK)])

    return k(idx3, table_scaled)


def kernel(tokens, table):
    b_total = tokens.shape[0] * tokens.shape[1]
    n_chunks = b_total // (NW * CHUNK)
    idx3 = tokens.reshape(NW, n_chunks, CHUNK).astype(jnp.int32)
    table_scaled = _scale_table(table)
    out = _sc_gather(idx3, table_scaled, n_chunks=n_chunks)
    return out.reshape(tokens.shape[0], tokens.shape[1], EMB)


# speedup vs baseline: 4.1608x; 1.2008x over previous
import functools
import math

import jax
import jax.numpy as jnp
from jax import lax
from jax.experimental import pallas as pl
from jax.experimental.pallas import tpu as pltpu
from jax.experimental.pallas import tpu_sc as plsc

EMB = 64
SCALE = math.sqrt(EMB)

NC = 2
NS = 16
NW = NC * NS

CHUNK = 128


def _scale_body(x_ref, o_ref):
    o_ref[...] = x_ref[...] * SCALE


def _scale_table(table):
    v, d = table.shape
    n = v * d
    cols = 128
    rows = n // cols
    block_rows = rows // 10
    t2 = table.reshape(rows, cols)
    scaled = pl.pallas_call(
        _scale_body,
        out_shape=jax.ShapeDtypeStruct((rows, cols), jnp.float32),
        grid=(10,),
        in_specs=[pl.BlockSpec((block_rows, cols), lambda i: (i, 0))],
        out_specs=pl.BlockSpec((block_rows, cols), lambda i: (i, 0)),
    )(t2)
    return scaled.reshape(v, d)


NBUF = 8
G = 4
O = 4


@functools.partial(jax.jit, static_argnames=("n_chunks",))
def _sc_gather(idx3, table_scaled, *, n_chunks):
    assert n_chunks >= 2 * NBUF and (n_chunks - NBUF) % NBUF == 0
    b_total = NW * n_chunks * CHUNK
    b_per_w = n_chunks * CHUNK
    mesh = plsc.VectorSubcoreMesh(core_axis_name="c", subcore_axis_name="s")

    @functools.partial(
        pl.kernel,
        out_type=jax.ShapeDtypeStruct((b_total, EMB), jnp.float32),
        mesh=mesh,
        scratch_types=(
            [pltpu.VMEM((n_chunks, CHUNK), jnp.int32),
             pltpu.VMEM((NBUF, CHUNK, EMB), jnp.float32)]
            + [pltpu.SemaphoreType.DMA] * (2 * NBUF)
        ),
        compiler_params=pltpu.CompilerParams(use_tc_tiling_on_sc=False),
    )
    def k(idx_hbm, tab_hbm, out_hbm, idx_v, rows_v, *sems):
        gsems, osems = sems[:NBUF], sems[NBUF:]
        wid = lax.axis_index("s") * NC + lax.axis_index("c")
        base = wid * b_per_w
        pltpu.sync_copy(idx_hbm.at[wid], idx_v)

        def start_gather(j, b):
            pltpu.async_copy(tab_hbm.at[idx_v.at[j]], rows_v.at[b], gsems[b])

        def wait_gather(b):
            pltpu.make_async_copy(
                tab_hbm.at[idx_v.at[0]], rows_v.at[b], gsems[b]).wait()

        def start_out(j, b):
            pltpu.async_copy(
                rows_v.at[b], out_hbm.at[pl.ds(base + j * CHUNK, CHUNK)],
                osems[b])

        def wait_out(b):
            pltpu.make_async_copy(
                rows_v.at[b], out_hbm.at[pl.ds(base, CHUNK)], osems[b]).wait()

        for b in range(G):
            start_gather(b, b)
        for j in range(O):
            wait_gather(j % NBUF)
            start_out(j, j % NBUF)
            start_gather(j + G, (j + G) % NBUF)

        @pl.loop(O, n_chunks - G, step=NBUF)
        def _blk(j0):
            for i in range(NBUF):
                j = j0 + i
                b = (O + i) % NBUF
                bf = i
                wait_gather(b)
                start_out(j, b)
                wait_out(bf)
                start_gather(j + G, bf)

        for j in range(n_chunks - G, n_chunks):
            wait_gather(j % NBUF)
            start_out(j, j % NBUF)
            wait_out((j + G) % NBUF)
        for j in range(n_chunks - O, n_chunks):
            wait_out(j % NBUF)

    return k(idx3, table_scaled)


def kernel(tokens, table):
    b_total = tokens.shape[0] * tokens.shape[1]
    n_chunks = b_total // (NW * CHUNK)
    idx3 = tokens.reshape(NW, n_chunks, CHUNK).astype(jnp.int32)
    table_scaled = _scale_table(table)
    out = _sc_gather(idx3, table_scaled, n_chunks=n_chunks)
    return out.reshape(tokens.shape[0], tokens.shape[1], EMB)
